# Initial kernel scaffold; baseline (speedup 1.0000x reference)
import jax
import jax.numpy as jnp
from jax.experimental import pallas as pl


def kernel(x, edge_index):
    raise NotImplementedError("write your pallas kernel here")


# speedup vs baseline: 4.9145x; 4.9145x over previous
import functools

import jax
import jax.numpy as jnp
from jax import lax
from jax.experimental import pallas as pl
from jax.experimental.pallas import tpu as pltpu
from jax.experimental.pallas import tpu_sc as plsc

N = 10000
D = 128
E = 320000
NC = 2
NS = 16
NW = NC * NS
EW = E // NW
CHUNK = 80
NCHUNK = EW // CHUNK
ROWS_MAIN = 624
TAIL_BASE = ROWS_MAIN * NS
TAIL = N - TAIL_BASE


def _spmm_body(h_hbm, col_hbm, row_hbm, zeros_hbm, out_hbm,
               colv, rowv, rows_v, accum, sem):
    cid = lax.axis_index("c")
    sid = lax.axis_index("s")
    wid = sid * NC + cid
    pltpu.sync_copy(zeros_hbm.at[pl.ds(0, ROWS_MAIN)],
                    accum.at[pl.ds(sid * ROWS_MAIN, ROWS_MAIN)])

    @pl.when(sid == NS - 1)
    def _zero_tail():
        pltpu.sync_copy(zeros_hbm.at[pl.ds(0, TAIL)],
                        accum.at[pl.ds(TAIL_BASE, TAIL)])

    plsc.subcore_barrier()

    ebase = wid * EW

    def chunk_step(i, carry):
        base = pl.multiple_of(ebase + i * CHUNK, 8)
        pltpu.sync_copy(col_hbm.at[pl.ds(base, CHUNK)], colv)
        pltpu.sync_copy(row_hbm.at[pl.ds(base, CHUNK)], rowv)
        pltpu.async_copy(h_hbm.at[colv], rows_v, sem).wait()
        pltpu.sync_copy(rows_v, accum.at[rowv], add=True)
        return carry

    lax.fori_loop(0, NCHUNK, chunk_step, 0)
    plsc.subcore_barrier()
    pltpu.sync_copy(accum.at[pl.ds(sid * ROWS_MAIN, ROWS_MAIN)],
                    out_hbm.at[cid, pl.ds(sid * ROWS_MAIN, ROWS_MAIN)])

    @pl.when(sid == NS - 1)
    def _export_tail():
        pltpu.sync_copy(accum.at[pl.ds(TAIL_BASE, TAIL)],
                        out_hbm.at[cid, pl.ds(TAIL_BASE, TAIL)])


_spmm = pl.kernel(
    _spmm_body,
    mesh=plsc.VectorSubcoreMesh(core_axis_name="c", subcore_axis_name="s"),
    out_type=jax.ShapeDtypeStruct((NC, N, D), jnp.float32),
    scratch_types=[
        pltpu.VMEM((CHUNK,), jnp.int32),
        pltpu.VMEM((CHUNK,), jnp.int32),
        pltpu.VMEM((CHUNK, D), jnp.float32),
        pltpu.VMEM_SHARED((N, D), jnp.float32),
        pltpu.SemaphoreType.DMA,
    ],
)

RB = 1000
GRID = N // RB


def _combine_body(include_input, p0, p1, h, outh, csum):
    s = p0[...] + p1[...]
    rolled = jnp.roll(s, 1, axis=1)
    hn = jnp.sign(rolled + h[...])
    outh[...] = hn
    part = jnp.sum(hn, axis=0, keepdims=True)
    if include_input:
        part = part + jnp.sum(h[...], axis=0, keepdims=True)

    @pl.when(pl.program_id(0) == 0)
    def _init():
        csum[...] = part

    @pl.when(pl.program_id(0) != 0)
    def _acc():
        csum[...] = csum[...] + part


def _make_combine(include_input):
    return pl.pallas_call(
        functools.partial(_combine_body, include_input),
        grid=(GRID,),
        in_specs=[pl.BlockSpec((RB, D), lambda i: (i, 0))] * 3,
        out_specs=[pl.BlockSpec((RB, D), lambda i: (i, 0)),
                   pl.BlockSpec((1, D), lambda i: (0, 0))],
        out_shape=[jax.ShapeDtypeStruct((N, D), jnp.float32),
                   jax.ShapeDtypeStruct((1, D), jnp.float32)],
    )


_combine_first = _make_combine(True)
_combine_rest = _make_combine(False)


def kernel(x, edge_index):
    row = edge_index[0]
    col = edge_index[1]
    zeros = jnp.zeros((ROWS_MAIN, D), jnp.float32)
    h = x
    total = None
    for layer in range(3):
        partials = _spmm(h, col, row, zeros)
        combine = _combine_first if layer == 0 else _combine_rest
        h, csum = combine(partials[0], partials[1], h)
        total = csum if total is None else total + csum
    return total.reshape(D)


# speedup vs baseline: 12.0146x; 2.4447x over previous
import functools

import jax
import jax.numpy as jnp
from jax import lax
from jax.experimental import pallas as pl
from jax.experimental.pallas import tpu as pltpu
from jax.experimental.pallas import tpu_sc as plsc

N = 10000
D = 128
E = 320000
NC = 2
NS = 16
NW = NC * NS
EW = E // NW
CHUNK = 125
NCHUNK = EW // CHUNK
NBUF = 2
ROWS_MAIN = 624
TAIL_BASE = ROWS_MAIN * NS
TAIL = N - TAIL_BASE


def _spmm_body(h_hbm, col_hbm, row_hbm, zeros_hbm, out_hbm,
               colv, ridx, *rest):
    bufs = rest[:NBUF]
    accum = rest[NBUF]
    gsems = rest[NBUF + 1:2 * NBUF + 1]
    rsems = rest[2 * NBUF + 1:]
    cid = lax.axis_index("c")
    sid = lax.axis_index("s")
    wid = sid * NC + cid
    pltpu.sync_copy(zeros_hbm.at[pl.ds(0, ROWS_MAIN)],
                    accum.at[pl.ds(sid * ROWS_MAIN, ROWS_MAIN)])

    @pl.when(sid == NS - 1)
    def _zero_tail():
        pltpu.sync_copy(zeros_hbm.at[pl.ds(0, TAIL)],
                        accum.at[pl.ds(TAIL_BASE, TAIL)])

    pltpu.sync_copy(col_hbm.at[wid], colv)

    def gather_copy(chunk, b):
        return pltpu.make_async_copy(
            h_hbm.at[colv.at[chunk]], bufs[b], gsems[b])

    def ridx_copy(chunk, b):
        return pltpu.make_async_copy(row_hbm.at[wid, chunk], ridx.at[b],
                                     rsems[b])

    plsc.subcore_barrier()

    for b in range(NBUF):
        gather_copy(b, b).start()
        ridx_copy(b, b).start()

    def outer_step(jj, carry):
        for b in range(NBUF):
            chunk = jj * NBUF + b
            gather_copy(chunk, b).wait()
            ridx_copy(chunk, b).wait()
            pltpu.sync_copy(bufs[b], accum.at[ridx.at[b]], add=True)

            @pl.when(jj < NCHUNK // NBUF - 1)
            def _prefetch():
                gather_copy(chunk + NBUF, b).start()
                ridx_copy(chunk + NBUF, b).start()

        return carry

    lax.fori_loop(0, NCHUNK // NBUF, outer_step, 0)
    plsc.subcore_barrier()
    pltpu.sync_copy(accum.at[pl.ds(sid * ROWS_MAIN, ROWS_MAIN)],
                    out_hbm.at[cid, pl.ds(sid * ROWS_MAIN, ROWS_MAIN)])

    @pl.when(sid == NS - 1)
    def _export_tail():
        pltpu.sync_copy(accum.at[pl.ds(TAIL_BASE, TAIL)],
                        out_hbm.at[cid, pl.ds(TAIL_BASE, TAIL)])


_spmm = pl.kernel(
    _spmm_body,
    mesh=plsc.VectorSubcoreMesh(core_axis_name="c", subcore_axis_name="s"),
    out_type=jax.ShapeDtypeStruct((NC, N, D), jnp.float32),
    scratch_types=(
        [pltpu.VMEM((NCHUNK, CHUNK), jnp.int32),
         pltpu.VMEM((NBUF, CHUNK), jnp.int32)]
        + [pltpu.VMEM((CHUNK, D), jnp.float32)] * NBUF
        + [pltpu.VMEM_SHARED((N, D), jnp.float32)]
        + [pltpu.SemaphoreType.DMA] * (2 * NBUF)
    ),
)

RB = 1000
GRID = N // RB


def _combine_body(include_input, p0, p1, h, outh, csum):
    s = p0[...] + p1[...]
    rolled = jnp.roll(s, 1, axis=1)
    hn = jnp.sign(rolled + h[...])
    outh[...] = hn
    part = jnp.sum(hn, axis=0, keepdims=True)
    if include_input:
        part = part + jnp.sum(h[...], axis=0, keepdims=True)

    @pl.when(pl.program_id(0) == 0)
    def _init():
        csum[...] = part

    @pl.when(pl.program_id(0) != 0)
    def _acc():
        csum[...] = csum[...] + part


def _make_combine(include_input):
    return pl.pallas_call(
        functools.partial(_combine_body, include_input),
        grid=(GRID,),
        in_specs=[pl.BlockSpec((RB, D), lambda i: (i, 0))] * 3,
        out_specs=[pl.BlockSpec((RB, D), lambda i: (i, 0)),
                   pl.BlockSpec((1, D), lambda i: (0, 0))],
        out_shape=[jax.ShapeDtypeStruct((N, D), jnp.float32),
                   jax.ShapeDtypeStruct((1, D), jnp.float32)],
    )


_combine_first = _make_combine(True)
_combine_rest = _make_combine(False)


def kernel(x, edge_index):
    row = edge_index[0].reshape(NW, NCHUNK, CHUNK)
    col = edge_index[1].reshape(NW, NCHUNK, CHUNK)
    zeros = jnp.zeros((ROWS_MAIN, D), jnp.float32)
    h = x
    total = None
    for layer in range(3):
        partials = _spmm(h, col, row, zeros)
        combine = _combine_first if layer == 0 else _combine_rest
        h, csum = combine(partials[0], partials[1], h)
        total = csum if total is None else total + csum
    return total.reshape(D)
